# Initial kernel scaffold; baseline (speedup 1.0000x reference)
import jax
import jax.numpy as jnp
from jax.experimental import pallas as pl


def kernel(x, edge_index, W1, b1, W2, b2):
    raise NotImplementedError("write your pallas kernel here")


# speedup vs baseline: 12.8058x; 12.8058x over previous
import functools

import jax
import jax.numpy as jnp
from jax import lax
from jax.experimental import pallas as pl
from jax.experimental.pallas import tpu as pltpu
from jax.experimental.pallas import tpu_sc as plsc

N = 10000
E = 320000
D = 128
NP = 10240
NC = 2
NS = 16
NT = NC * NS
ET = E // NT
CH = 80
NCH = ET // CH
RPT = NP // NS
BLK = 512


def _sc_degree_body(dst_hbm, out_hbm, didx, ones, zb, dga):
    cid = lax.axis_index("c")
    sid = lax.axis_index("s")
    ebase = (cid * NS + sid) * ET
    rbase = sid * RPT
    one = jnp.ones((16,), jnp.float32)
    zero = jnp.zeros((16,), jnp.float32)

    def fill_ones(i, c):
        ones[pl.ds(i * 16, 16)] = one
        return c

    lax.fori_loop(0, CH // 16, fill_ones, 0)

    def fill_zero(i, c):
        zb[pl.ds(i * 16, 16)] = zero
        return c

    lax.fori_loop(0, RPT // 16, fill_zero, 0)
    pltpu.sync_copy(zb, dga.at[pl.ds(rbase, RPT)])
    plsc.subcore_barrier()

    def step(j, c):
        base = ebase + j * CH
        pltpu.sync_copy(dst_hbm.at[pl.ds(base, CH)], didx.at[0])
        pltpu.sync_copy(ones, dga.at[didx.at[0]], add=True)
        return c

    lax.fori_loop(0, NCH, step, 0)
    plsc.subcore_barrier()
    pltpu.sync_copy(dga.at[pl.ds(rbase, RPT)], out_hbm.at[cid, pl.ds(rbase, RPT)])


def _sc_scatter_body(y_hbm, src_hbm, dst_hbm, out_hbm, sidx, didx, rows, zbuf, acc, sem):
    cid = lax.axis_index("c")
    sid = lax.axis_index("s")
    ebase = (cid * NS + sid) * ET
    rbase = sid * RPT
    zero = jnp.zeros((16,), jnp.float32)

    def fill_zero(i, c):
        zbuf[i // (D // 16), pl.ds((i % (D // 16)) * 16, 16)] = zero
        return c

    lax.fori_loop(0, CH * (D // 16), fill_zero, 0)
    for k in range(RPT // CH):
        pltpu.sync_copy(zbuf, acc.at[pl.ds(rbase + k * CH, CH)])
    plsc.subcore_barrier()

    def step(j, c):
        base = ebase + j * CH
        pltpu.sync_copy(src_hbm.at[pl.ds(base, CH)], sidx.at[0])
        pltpu.sync_copy(dst_hbm.at[pl.ds(base, CH)], didx.at[0])
        pltpu.async_copy(y_hbm.at[sidx.at[0]], rows, sem).wait()
        pltpu.sync_copy(rows, acc.at[didx.at[0]], add=True)
        return c

    lax.fori_loop(0, NCH, step, 0)
    plsc.subcore_barrier()
    pltpu.sync_copy(acc.at[pl.ds(rbase, RPT)], out_hbm.at[cid, pl.ds(rbase, RPT)])


@functools.cache
def _sc_kernels():
    mesh = plsc.VectorSubcoreMesh(
        core_axis_name="c", subcore_axis_name="s", num_cores=NC, num_subcores=NS
    )
    degree = pl.kernel(
        _sc_degree_body,
        out_type=jax.ShapeDtypeStruct((NC, NP), jnp.float32),
        mesh=mesh,
        scratch_types=[
            pltpu.VMEM((1, CH), jnp.int32),
            pltpu.VMEM((CH,), jnp.float32),
            pltpu.VMEM((RPT,), jnp.float32),
            pltpu.VMEM_SHARED((NP,), jnp.float32),
        ],
    )
    scatter = pl.kernel(
        _sc_scatter_body,
        out_type=jax.ShapeDtypeStruct((NC, NP, D), jnp.float32),
        mesh=mesh,
        scratch_types=[
            pltpu.VMEM((1, CH), jnp.int32),
            pltpu.VMEM((1, CH), jnp.int32),
            pltpu.VMEM((CH, D), jnp.float32),
            pltpu.VMEM((CH, D), jnp.float32),
            pltpu.VMEM_SHARED((NP, D), jnp.float32),
            pltpu.SemaphoreType.DMA,
        ],
    )
    return degree, scatter


def _y_body(x_ref, dg_ref, w_ref, o_ref):
    dinv = lax.rsqrt(dg_ref[0] + dg_ref[1] + 1.0)
    xw = jnp.dot(x_ref[...], w_ref[...], preferred_element_type=jnp.float32)
    o_ref[...] = xw * dinv


def _mid_body(acc_ref, y_ref, dg_ref, b_ref, w_ref, o_ref):
    dinv = lax.rsqrt(dg_ref[0] + dg_ref[1] + 1.0)
    s = (acc_ref[0] + acc_ref[1] + y_ref[...]) * dinv + b_ref[...]
    h = jnp.maximum(s, 0.0)
    hw = jnp.dot(h, w_ref[...], preferred_element_type=jnp.float32)
    o_ref[...] = hw * dinv


def _fin_body(acc_ref, y_ref, dg_ref, b_ref, o_ref):
    dinv = lax.rsqrt(dg_ref[0] + dg_ref[1] + 1.0)
    o_ref[...] = (acc_ref[0] + acc_ref[1] + y_ref[...]) * dinv + b_ref[...]


_dg_spec = pl.BlockSpec((NC, BLK, 1), lambda i: (0, i, 0))
_row_spec = pl.BlockSpec((BLK, D), lambda i: (i, 0))
_acc_spec = pl.BlockSpec((NC, BLK, D), lambda i: (0, i, 0))
_w_spec = pl.BlockSpec((D, D), lambda i: (0, 0))
_b_spec = pl.BlockSpec((1, D), lambda i: (0, 0))
_grid = (NP // BLK,)
_row_out = jax.ShapeDtypeStruct((NP, D), jnp.float32)

_tc_y = pl.pallas_call(
    _y_body,
    grid=_grid,
    in_specs=[_row_spec, _dg_spec, _w_spec],
    out_specs=_row_spec,
    out_shape=_row_out,
)

_tc_mid = pl.pallas_call(
    _mid_body,
    grid=_grid,
    in_specs=[_acc_spec, _row_spec, _dg_spec, _b_spec, _w_spec],
    out_specs=_row_spec,
    out_shape=_row_out,
)

_tc_fin = pl.pallas_call(
    _fin_body,
    grid=_grid,
    in_specs=[_acc_spec, _row_spec, _dg_spec, _b_spec],
    out_specs=_row_spec,
    out_shape=_row_out,
)


def kernel(x, edge_index, W1, b1, W2, b2):
    sc_degree, sc_scatter = _sc_kernels()
    src = edge_index[0]
    dst = edge_index[1]
    degp = sc_degree(dst)
    deg3 = degp.reshape(NC, NP, 1)
    xp = jnp.pad(x, ((0, NP - N), (0, 0)))
    b1r = b1.reshape(1, D)
    b2r = b2.reshape(1, D)
    y1 = _tc_y(xp, deg3, W1)
    acc1 = sc_scatter(y1, src, dst)
    y2 = _tc_mid(acc1, y1, deg3, b1r, W2)
    acc2 = sc_scatter(y2, src, dst)
    out = _tc_fin(acc2, y2, deg3, b2r)
    return out[:N]


# speedup vs baseline: 23.4528x; 1.8314x over previous
import functools

import jax
import jax.numpy as jnp
from jax import lax
from jax.experimental import pallas as pl
from jax.experimental.pallas import tpu as pltpu
from jax.experimental.pallas import tpu_sc as plsc

N = 10000
E = 320000
D = 128
NP = 10240
NC = 2
NS = 16
NT = NC * NS
ET = E // NT
CH = 80
NCH = ET // CH
RPT = NP // NS
BLK = 512


def _sc_degree_body(dst_hbm, out_hbm, didx, ones, zb, dga):
    cid = lax.axis_index("c")
    sid = lax.axis_index("s")
    wid = cid * NS + sid
    rbase = sid * RPT
    one = jnp.ones((16,), jnp.float32)
    zero = jnp.zeros((16,), jnp.float32)

    def fill_ones(i, c):
        ones[pl.ds(i * 16, 16)] = one
        return c

    lax.fori_loop(0, CH // 16, fill_ones, 0)

    def fill_zero(i, c):
        zb[pl.ds(i * 16, 16)] = zero
        return c

    lax.fori_loop(0, RPT // 16, fill_zero, 0)
    pltpu.sync_copy(dst_hbm.at[wid], didx)
    pltpu.sync_copy(zb, dga.at[pl.ds(rbase, RPT)])
    plsc.subcore_barrier()

    def step(j, c):
        pltpu.sync_copy(ones, dga.at[didx.at[j]], add=True)
        return c

    lax.fori_loop(0, NCH, step, 0)
    plsc.subcore_barrier()
    pltpu.sync_copy(dga.at[pl.ds(rbase, RPT)], out_hbm.at[cid, pl.ds(rbase, RPT)])


def _sc_scatter_body(y_hbm, src_hbm, dst_hbm, out_hbm, sidx, didx, rows, acc, sem):
    cid = lax.axis_index("c")
    sid = lax.axis_index("s")
    wid = cid * NS + sid
    rbase = sid * RPT
    zero = jnp.zeros((16,), jnp.float32)

    def fill_zero(i, c):
        rows[1, i // (D // 16), pl.ds((i % (D // 16)) * 16, 16)] = zero
        return c

    lax.fori_loop(0, CH * (D // 16), fill_zero, 0)
    pltpu.sync_copy(src_hbm.at[pl.ds(wid * ET, ET)], sidx)
    pltpu.sync_copy(dst_hbm.at[wid], didx)
    for k in range(RPT // CH):
        pltpu.sync_copy(rows.at[1], acc.at[pl.ds(rbase + k * CH, CH)])
    plsc.subcore_barrier()

    pltpu.async_copy(y_hbm.at[sidx.at[pl.ds(0, CH)]], rows.at[0], sem)

    def step(j, c):
        b = lax.rem(j, 2)
        idx_j = sidx.at[pl.ds(j * CH, CH)]
        pltpu.make_async_copy(y_hbm.at[idx_j], rows.at[b], sem).wait()

        @pl.when(j + 1 < NCH)
        def _():
            idx_n = sidx.at[pl.ds((j + 1) * CH, CH)]
            pltpu.async_copy(y_hbm.at[idx_n], rows.at[1 - b], sem)

        pltpu.sync_copy(rows.at[b], acc.at[didx.at[j]], add=True)
        return c

    lax.fori_loop(0, NCH, step, 0)
    plsc.subcore_barrier()
    pltpu.sync_copy(acc.at[pl.ds(rbase, RPT)], out_hbm.at[cid, pl.ds(rbase, RPT)])


@functools.cache
def _sc_kernels():
    mesh = plsc.VectorSubcoreMesh(
        core_axis_name="c", subcore_axis_name="s", num_cores=NC, num_subcores=NS
    )
    degree = pl.kernel(
        _sc_degree_body,
        out_type=jax.ShapeDtypeStruct((NC, NP), jnp.float32),
        mesh=mesh,
        scratch_types=[
            pltpu.VMEM((NCH, CH), jnp.int32),
            pltpu.VMEM((CH,), jnp.float32),
            pltpu.VMEM((RPT,), jnp.float32),
            pltpu.VMEM_SHARED((NP,), jnp.float32),
        ],
    )
    scatter = pl.kernel(
        _sc_scatter_body,
        out_type=jax.ShapeDtypeStruct((NC, NP, D), jnp.float32),
        mesh=mesh,
        scratch_types=[
            pltpu.VMEM((ET,), jnp.int32),
            pltpu.VMEM((NCH, CH), jnp.int32),
            pltpu.VMEM((2, CH, D), jnp.float32),
            pltpu.VMEM_SHARED((NP, D), jnp.float32),
            pltpu.SemaphoreType.DMA,
        ],
    )
    return degree, scatter


def _y_body(x_ref, dg_ref, w_ref, o_ref):
    dinv = lax.rsqrt(dg_ref[0] + dg_ref[1] + 1.0)
    xw = jnp.dot(x_ref[...], w_ref[...], preferred_element_type=jnp.float32)
    o_ref[...] = xw * dinv


def _mid_body(acc_ref, y_ref, dg_ref, b_ref, w_ref, o_ref):
    dinv = lax.rsqrt(dg_ref[0] + dg_ref[1] + 1.0)
    s = (acc_ref[0] + acc_ref[1] + y_ref[...]) * dinv + b_ref[...]
    h = jnp.maximum(s, 0.0)
    hw = jnp.dot(h, w_ref[...], preferred_element_type=jnp.float32)
    o_ref[...] = hw * dinv


def _fin_body(acc_ref, y_ref, dg_ref, b_ref, o_ref):
    dinv = lax.rsqrt(dg_ref[0] + dg_ref[1] + 1.0)
    o_ref[...] = (acc_ref[0] + acc_ref[1] + y_ref[...]) * dinv + b_ref[...]


_dg_spec = pl.BlockSpec((NC, BLK, 1), lambda i: (0, i, 0))
_row_spec = pl.BlockSpec((BLK, D), lambda i: (i, 0))
_acc_spec = pl.BlockSpec((NC, BLK, D), lambda i: (0, i, 0))
_w_spec = pl.BlockSpec((D, D), lambda i: (0, 0))
_b_spec = pl.BlockSpec((1, D), lambda i: (0, 0))
_grid = (NP // BLK,)
_row_out = jax.ShapeDtypeStruct((NP, D), jnp.float32)

_tc_y = pl.pallas_call(
    _y_body,
    grid=_grid,
    in_specs=[_row_spec, _dg_spec, _w_spec],
    out_specs=_row_spec,
    out_shape=_row_out,
)

_tc_mid = pl.pallas_call(
    _mid_body,
    grid=_grid,
    in_specs=[_acc_spec, _row_spec, _dg_spec, _b_spec, _w_spec],
    out_specs=_row_spec,
    out_shape=_row_out,
)

_tc_fin = pl.pallas_call(
    _fin_body,
    grid=_grid,
    in_specs=[_acc_spec, _row_spec, _dg_spec, _b_spec],
    out_specs=_row_spec,
    out_shape=_row_out,
)


def kernel(x, edge_index, W1, b1, W2, b2):
    sc_degree, sc_scatter = _sc_kernels()
    src = edge_index[0]
    dst = edge_index[1].reshape(NT, NCH, CH)
    degp = sc_degree(dst)
    deg3 = degp.reshape(NC, NP, 1)
    xp = jnp.pad(x, ((0, NP - N), (0, 0)))
    b1r = b1.reshape(1, D)
    b2r = b2.reshape(1, D)
    y1 = _tc_y(xp, deg3, W1)
    acc1 = sc_scatter(y1, src, dst)
    y2 = _tc_mid(acc1, y1, deg3, b1r, W2)
    acc2 = sc_scatter(y2, src, dst)
    out = _tc_fin(acc2, y2, deg3, b2r)
    return out[:N]
